# baseline (speedup 1.0000x reference)
import functools

import jax
import jax.numpy as jnp
from jax import lax
from jax.experimental import pallas as pl
from jax.experimental.pallas import tpu as pltpu
from jax.experimental.pallas import tpu_sc as plsc

N_TYPES = 3
EMB_D = 64
PAIR = 2
PD = EMB_D * PAIR
PCHUNK = 128


@functools.lru_cache(maxsize=None)
def _make_lookup(bp: int):
    info = plsc.get_sparse_core_info()
    nw = info.num_cores * info.num_subcores
    per_w = bp // nw
    n_chunks = per_w // PCHUNK
    assert bp % (nw * PCHUNK) == 0 and n_chunks % 2 == 0

    mesh = plsc.VectorSubcoreMesh(core_axis_name="c", subcore_axis_name="s")

    @functools.partial(
        pl.kernel,
        mesh=mesh,
        out_type=jax.ShapeDtypeStruct((bp, PD), jnp.float32),
        scratch_types=[
            pltpu.VMEM((n_chunks, PCHUNK), jnp.int32),
            pltpu.VMEM((PCHUNK, PD), jnp.float32),
            pltpu.VMEM((PCHUNK, PD), jnp.float32),
            pltpu.SemaphoreType.DMA,
            pltpu.SemaphoreType.DMA,
            pltpu.SemaphoreType.DMA,
            pltpu.SemaphoreType.DMA,
        ],
    )
    def lookup(ptbl_hbm, pid_hbm, out_hbm, pid_v, rows0, rows1,
               gsem0, gsem1, osem0, osem1):
        wid = lax.axis_index("s") * info.num_cores + lax.axis_index("c")
        base0 = wid * per_w

        pltpu.sync_copy(pid_hbm.at[pl.ds(wid * n_chunks, n_chunks)], pid_v)

        def gather(c, rows, gsem):
            pltpu.async_copy(ptbl_hbm.at[pid_v.at[c]], rows, gsem)

        def wait_gather(c, rows, gsem):
            pltpu.make_async_copy(ptbl_hbm.at[pid_v.at[c]], rows, gsem).wait()

        def put(c, rows, osem):
            pltpu.async_copy(
                rows, out_hbm.at[pl.ds(base0 + c * PCHUNK, PCHUNK)], osem)

        def wait_put(c, rows, osem):
            pltpu.make_async_copy(
                rows, out_hbm.at[pl.ds(base0 + c * PCHUNK, PCHUNK)], osem).wait()

        gather(0, rows0, gsem0)
        gather(1, rows1, gsem1)

        def body(j, carry):
            c0 = 2 * j
            wait_gather(c0, rows0, gsem0)
            put(c0, rows0, osem0)
            wait_gather(c0 + 1, rows1, gsem1)
            put(c0 + 1, rows1, osem1)
            wait_put(c0, rows0, osem0)

            @pl.when(c0 + 2 < n_chunks)
            def _():
                gather(c0 + 2, rows0, gsem0)

            wait_put(c0 + 1, rows1, osem1)

            @pl.when(c0 + 3 < n_chunks)
            def _():
                gather(c0 + 3, rows1, gsem1)

            return carry

        lax.fori_loop(0, n_chunks // 2, body, 0)

    return lookup


def kernel(type_indices, embedding_weight):
    b, s = type_indices.shape
    pairs = type_indices.reshape(b * s // PAIR, PAIR).astype(jnp.int32)
    pid = (pairs[:, 0] * N_TYPES + pairs[:, 1]).reshape(-1, PCHUNK)
    p = jnp.arange(N_TYPES**PAIR)
    ptbl = jnp.concatenate(
        [embedding_weight[p // N_TYPES], embedding_weight[p % N_TYPES]], axis=-1)
    out = _make_lookup(b * s // PAIR)(ptbl, pid)
    return out.reshape(b, s, EMB_D)


# speedup vs baseline: 2.7092x; 2.7092x over previous
import functools

import jax
import jax.numpy as jnp
from jax import lax
from jax.experimental import pallas as pl
from jax.experimental.pallas import tpu as pltpu
from jax.experimental.pallas import tpu_sc as plsc

N_TYPES = 3
EMB_D = 64
QUAD = 4
QD = EMB_D * QUAD
QCHUNK = 128
ICHUNK = QCHUNK * QUAD


@functools.lru_cache(maxsize=None)
def _make_lookup(bq: int):
    info = plsc.get_sparse_core_info()
    nw = info.num_cores * info.num_subcores
    per_w = bq // nw
    n_chunks = per_w // QCHUNK
    assert bq % (nw * QCHUNK) == 0 and n_chunks % 2 == 0

    mesh = plsc.VectorSubcoreMesh(core_axis_name="c", subcore_axis_name="s")

    @functools.partial(
        pl.kernel,
        mesh=mesh,
        out_type=jax.ShapeDtypeStruct((bq, QD), jnp.float32),
        scratch_types=[
            pltpu.VMEM((per_w,), jnp.int32),
            pltpu.VMEM((QCHUNK,), jnp.int32),
            pltpu.VMEM((QCHUNK,), jnp.int32),
            pltpu.VMEM((QCHUNK, QD), jnp.float32),
            pltpu.VMEM((QCHUNK, QD), jnp.float32),
            pltpu.SemaphoreType.DMA,
            pltpu.SemaphoreType.DMA,
            pltpu.SemaphoreType.DMA,
            pltpu.SemaphoreType.DMA,
        ],
    )
    def lookup(qtbl_hbm, idxq_hbm, out_hbm, idxq_v, qb0, qb1, rows0, rows1,
               gsem0, gsem1, osem0, osem1):
        wid = lax.axis_index("s") * info.num_cores + lax.axis_index("c")
        base0 = wid * per_w

        pltpu.sync_copy(idxq_hbm.at[pl.ds(base0, per_w)], idxq_v)

        def compute_qid(c, qb):
            for g in range(QCHUNK // 16):
                w = idxq_v[pl.ds(c * QCHUNK + 16 * g, 16)]
                a = w & 255
                b = lax.shift_right_logical(w, 8) & 255
                cc = lax.shift_right_logical(w, 16) & 255
                dd = lax.shift_right_logical(w, 24)
                qb[pl.ds(16 * g, 16)] = ((a * N_TYPES + b) * N_TYPES + cc) \
                    * N_TYPES + dd

        def gather(c, rows, qb, gsem):
            pltpu.async_copy(qtbl_hbm.at[qb], rows, gsem)

        def wait_gather(c, rows, qb, gsem):
            pltpu.make_async_copy(qtbl_hbm.at[qb], rows, gsem).wait()

        def put(c, rows, osem):
            pltpu.async_copy(
                rows, out_hbm.at[pl.ds(base0 + c * QCHUNK, QCHUNK)], osem)

        def wait_put(c, rows, osem):
            pltpu.make_async_copy(
                rows, out_hbm.at[pl.ds(base0 + c * QCHUNK, QCHUNK)], osem).wait()

        compute_qid(0, qb0)
        gather(0, rows0, qb0, gsem0)
        compute_qid(1, qb1)
        gather(1, rows1, qb1, gsem1)

        def body(j, carry):
            c0 = 2 * j
            wait_gather(c0, rows0, qb0, gsem0)
            put(c0, rows0, osem0)

            @pl.when(c0 + 2 < n_chunks)
            def _():
                compute_qid(c0 + 2, qb0)

            wait_gather(c0 + 1, rows1, qb1, gsem1)
            put(c0 + 1, rows1, osem1)

            @pl.when(c0 + 3 < n_chunks)
            def _():
                compute_qid(c0 + 3, qb1)

            wait_put(c0, rows0, osem0)

            @pl.when(c0 + 2 < n_chunks)
            def _():
                gather(c0 + 2, rows0, qb0, gsem0)

            wait_put(c0 + 1, rows1, osem1)

            @pl.when(c0 + 3 < n_chunks)
            def _():
                gather(c0 + 3, rows1, qb1, gsem1)

            return carry

        lax.fori_loop(0, n_chunks // 2, body, 0)

    return lookup


def kernel(type_indices, embedding_weight):
    b, s = type_indices.shape
    idx8 = type_indices.reshape(b * s // QUAD, QUAD).astype(jnp.int8)
    idxq = lax.bitcast_convert_type(idx8, jnp.int32)
    q = jnp.arange(N_TYPES**QUAD)
    digits = jnp.stack(
        [(q // (N_TYPES**(QUAD - 1 - k))) % N_TYPES for k in range(QUAD)], axis=-1
    )
    qtbl = embedding_weight[digits].reshape(N_TYPES**QUAD, QD)
    out = _make_lookup(b * s // QUAD)(qtbl, idxq)
    return out.reshape(b, s, EMB_D)


# speedup vs baseline: 2.7304x; 1.0078x over previous
import functools

import jax
import jax.numpy as jnp
from jax import lax
from jax.experimental import pallas as pl
from jax.experimental.pallas import tpu as pltpu
from jax.experimental.pallas import tpu_sc as plsc

N_TYPES = 3
EMB_D = 64
QUAD = 4
QD = EMB_D * QUAD
NQ = N_TYPES**QUAD


@functools.lru_cache(maxsize=None)
def _make_lookup(nb: int, s: int):
    info = plsc.get_sparse_core_info()
    nw = info.num_cores * info.num_subcores
    b_per_w = nb // nw
    qrow = s // QUAD
    assert nb % (2 * nw) == 0 and s % QUAD == 0

    mesh = plsc.VectorSubcoreMesh(core_axis_name="c", subcore_axis_name="s")

    @functools.partial(
        pl.kernel,
        mesh=mesh,
        compiler_params=pltpu.CompilerParams(use_tc_tiling_on_sc=False),
        out_type=jax.ShapeDtypeStruct((nb, s, EMB_D), jnp.float32),
        scratch_types=[
            pltpu.VMEM((NQ, QD), jnp.float32),
            pltpu.VMEM((s + 16,), jnp.int32),
            pltpu.VMEM((s + 16,), jnp.int32),
            pltpu.VMEM((s, EMB_D), jnp.float32),
            pltpu.VMEM((s, EMB_D), jnp.float32),
            pltpu.SemaphoreType.DMA,
            pltpu.SemaphoreType.DMA,
            pltpu.SemaphoreType.DMA,
            pltpu.SemaphoreType.DMA,
        ],
    )
    def lookup(qtbl_hbm, idx_hbm, out_hbm, qtbl_v, idxA, idxB, rowsA, rowsB,
               isemA, isemB, osemA, osemB):
        wid = lax.axis_index("s") * info.num_cores + lax.axis_index("c")
        bbase = wid * b_per_w

        pltpu.sync_copy(qtbl_hbm, qtbl_v)

        def fetch_idx(r, idx_v, isem):
            pltpu.async_copy(idx_hbm.at[bbase + r], idx_v.at[pl.ds(0, s)], isem)

        def wait_idx(r, idx_v, isem):
            pltpu.make_async_copy(
                idx_hbm.at[bbase + r], idx_v.at[pl.ds(0, s)], isem).wait()

        def put(r, rows, osem):
            pltpu.async_copy(rows, out_hbm.at[bbase + r], osem)

        def wait_put(r, rows, osem):
            pltpu.make_async_copy(rows, out_hbm.at[bbase + r], osem).wait()

        def fill(idx_v, rows):
            for g in range((qrow + 3) // 4):
                v = idx_v[pl.ds(16 * g, 16)]
                for q in range(min(4, qrow - 4 * g)):
                    k = 4 * g + q
                    qid = ((v[4 * q] * N_TYPES + v[4 * q + 1]) * N_TYPES
                           + v[4 * q + 2]) * N_TYPES + v[4 * q + 3]
                    for j in range(QD // 16):
                        rows[4 * k + j // 4, pl.ds(16 * (j % 4), 16)] = \
                            qtbl_v[qid, pl.ds(16 * j, 16)]

        fetch_idx(0, idxA, isemA)
        fetch_idx(1, idxB, isemB)

        def body(j, carry):
            r0 = 2 * j
            wait_idx(r0, idxA, isemA)

            @pl.when(r0 >= 2)
            def _():
                wait_put(r0 - 2, rowsA, osemA)

            fill(idxA, rowsA)
            put(r0, rowsA, osemA)

            @pl.when(r0 + 2 < b_per_w)
            def _():
                fetch_idx(r0 + 2, idxA, isemA)

            wait_idx(r0 + 1, idxB, isemB)

            @pl.when(r0 >= 2)
            def _():
                wait_put(r0 - 1, rowsB, osemB)

            fill(idxB, rowsB)
            put(r0 + 1, rowsB, osemB)

            @pl.when(r0 + 3 < b_per_w)
            def _():
                fetch_idx(r0 + 3, idxB, isemB)

            return carry

        lax.fori_loop(0, b_per_w // 2, body, 0)
        wait_put(b_per_w - 2, rowsA, osemA)
        wait_put(b_per_w - 1, rowsB, osemB)

    return lookup


def kernel(type_indices, embedding_weight):
    b, s = type_indices.shape
    q = jnp.arange(NQ)
    digits = jnp.stack(
        [(q // (N_TYPES**(QUAD - 1 - k))) % N_TYPES for k in range(QUAD)], axis=-1
    )
    qtbl = embedding_weight[digits].reshape(NQ, QD)
    return _make_lookup(b, s)(qtbl, type_indices.astype(jnp.int32))


# speedup vs baseline: 3.7374x; 1.3688x over previous
import functools

import jax
import jax.numpy as jnp
from jax import lax
from jax.experimental import pallas as pl
from jax.experimental.pallas import tpu as pltpu
from jax.experimental.pallas import tpu_sc as plsc

N_TYPES = 3
EMB_D = 64
LANES = 16


@functools.lru_cache(maxsize=None)
def _make_lookup(nb: int, s: int):
    info = plsc.get_sparse_core_info()
    nw = info.num_cores * info.num_subcores
    b_per_w = nb // nw
    ngrp = b_per_w // LANES
    assert nb % (nw * LANES) == 0 and s % 2 == 0

    mesh = plsc.VectorSubcoreMesh(core_axis_name="c", subcore_axis_name="s")

    @functools.partial(
        pl.kernel,
        mesh=mesh,
        compiler_params=pltpu.CompilerParams(use_tc_tiling_on_sc=False),
        out_type=jax.ShapeDtypeStruct((s, EMB_D, nb), jnp.float32),
        scratch_types=[
            pltpu.VMEM((N_TYPES, EMB_D, LANES), jnp.float32),
            pltpu.VMEM((b_per_w,), jnp.int32),
            pltpu.VMEM((b_per_w,), jnp.int32),
            pltpu.VMEM((EMB_D, b_per_w), jnp.float32),
            pltpu.VMEM((EMB_D, b_per_w), jnp.float32),
            pltpu.SemaphoreType.DMA,
            pltpu.SemaphoreType.DMA,
            pltpu.SemaphoreType.DMA,
            pltpu.SemaphoreType.DMA,
        ],
    )
    def lookup(wsplat_hbm, idxt_hbm, out_hbm, wsplat_v, idxA, idxB,
               rowsA, rowsB, isemA, isemB, osemA, osemB):
        wid = lax.axis_index("s") * info.num_cores + lax.axis_index("c")
        bbase = wid * b_per_w

        pltpu.sync_copy(wsplat_hbm, wsplat_v)

        def fetch_idx(p, idx_v, isem):
            pltpu.async_copy(idxt_hbm.at[p, pl.ds(bbase, b_per_w)], idx_v, isem)

        def wait_idx(p, idx_v, isem):
            pltpu.make_async_copy(
                idxt_hbm.at[p, pl.ds(bbase, b_per_w)], idx_v, isem).wait()

        def put(p, rows, osem):
            pltpu.async_copy(
                rows, out_hbm.at[p, pl.ds(0, EMB_D), pl.ds(bbase, b_per_w)],
                osem)

        def wait_put(p, rows, osem):
            pltpu.make_async_copy(
                rows, out_hbm.at[p, pl.ds(0, EMB_D), pl.ds(bbase, b_per_w)],
                osem).wait()

        def fill(idx_v, rows):
            def dbody(d, carry):
                w0 = wsplat_v[0, d]
                w1 = wsplat_v[1, d]
                w2 = wsplat_v[2, d]
                for g in range(ngrp):
                    v = idx_v[pl.ds(LANES * g, LANES)]
                    rows[d, pl.ds(LANES * g, LANES)] = jnp.where(
                        v == 0, w0, jnp.where(v == 1, w1, w2))
                return carry

            lax.fori_loop(0, EMB_D, dbody, 0)

        fetch_idx(0, idxA, isemA)
        fetch_idx(1, idxB, isemB)

        def body(j, carry):
            p0 = 2 * j
            wait_idx(p0, idxA, isemA)

            @pl.when(p0 >= 2)
            def _():
                wait_put(p0 - 2, rowsA, osemA)

            fill(idxA, rowsA)
            put(p0, rowsA, osemA)

            @pl.when(p0 + 2 < s)
            def _():
                fetch_idx(p0 + 2, idxA, isemA)

            wait_idx(p0 + 1, idxB, isemB)

            @pl.when(p0 >= 2)
            def _():
                wait_put(p0 - 1, rowsB, osemB)

            fill(idxB, rowsB)
            put(p0 + 1, rowsB, osemB)

            @pl.when(p0 + 3 < s)
            def _():
                fetch_idx(p0 + 3, idxB, isemB)

            return carry

        lax.fori_loop(0, s // 2, body, 0)
        wait_put(s - 2, rowsA, osemA)
        wait_put(s - 1, rowsB, osemB)

    return lookup


def kernel(type_indices, embedding_weight):
    b, s = type_indices.shape
    idxt = type_indices.T.astype(jnp.int32)
    wsplat = jnp.broadcast_to(
        embedding_weight[:, :, None], (N_TYPES, EMB_D, LANES))
    out_t = _make_lookup(b, s)(wsplat, idxt)
    return jnp.transpose(out_t, (2, 0, 1))


# speedup vs baseline: 7.0521x; 1.8869x over previous
import functools

import jax
import jax.numpy as jnp
from jax import lax
from jax.experimental import pallas as pl
from jax.experimental.pallas import tpu as pltpu
from jax.experimental.pallas import tpu_sc as plsc

N_TYPES = 3
EMB_D = 64
LANES = 16


@functools.lru_cache(maxsize=None)
def _make_lookup(nb: int, s: int):
    info = plsc.get_sparse_core_info()
    nw = info.num_cores * info.num_subcores
    b_per_w = nb // nw
    ngrp = b_per_w // LANES
    assert nb % (nw * LANES) == 0 and s % 2 == 0

    mesh = plsc.VectorSubcoreMesh(core_axis_name="c", subcore_axis_name="s")

    @functools.partial(
        pl.kernel,
        mesh=mesh,
        compiler_params=pltpu.CompilerParams(use_tc_tiling_on_sc=False),
        out_type=jax.ShapeDtypeStruct((s, EMB_D, nb), jnp.float32),
        scratch_types=[
            pltpu.VMEM((N_TYPES, EMB_D, LANES), jnp.float32),
            pltpu.VMEM((b_per_w,), jnp.int32),
            pltpu.VMEM((b_per_w,), jnp.int32),
            pltpu.VMEM((EMB_D, b_per_w), jnp.float32),
            pltpu.VMEM((EMB_D, b_per_w), jnp.float32),
            pltpu.SemaphoreType.DMA,
            pltpu.SemaphoreType.DMA,
            pltpu.SemaphoreType.DMA,
            pltpu.SemaphoreType.DMA,
        ],
    )
    def lookup(wsplat_hbm, idxt_hbm, out_hbm, wsplat_v, idxA, idxB,
               rowsA, rowsB, isemA, isemB, osemA, osemB):
        wid = lax.axis_index("s") * info.num_cores + lax.axis_index("c")
        bbase = wid * b_per_w

        pltpu.sync_copy(wsplat_hbm, wsplat_v)

        def fetch_idx(p, idx_v, isem):
            pltpu.async_copy(idxt_hbm.at[p, pl.ds(bbase, b_per_w)], idx_v, isem)

        def wait_idx(p, idx_v, isem):
            pltpu.make_async_copy(
                idxt_hbm.at[p, pl.ds(bbase, b_per_w)], idx_v, isem).wait()

        def put(p, rows, osem):
            pltpu.async_copy(
                rows, out_hbm.at[p, pl.ds(0, EMB_D), pl.ds(bbase, b_per_w)],
                osem)

        def wait_put(p, rows, osem):
            pltpu.make_async_copy(
                rows, out_hbm.at[p, pl.ds(0, EMB_D), pl.ds(bbase, b_per_w)],
                osem).wait()

        def fill(idx_v, rows):
            for gb in range(ngrp // 4):
                ms = []
                for i in range(4):
                    v = idx_v[pl.ds(LANES * (4 * gb + i), LANES)]
                    ms.append((v == 0, v == 1))

                def dbody(d, carry, gb=gb, ms=ms):
                    w0 = wsplat_v[0, d]
                    w1 = wsplat_v[1, d]
                    w2 = wsplat_v[2, d]
                    for i in range(4):
                        m0, m1 = ms[i]
                        rows[d, pl.ds(LANES * (4 * gb + i), LANES)] = \
                            jnp.where(m0, w0, jnp.where(m1, w1, w2))
                    return carry

                lax.fori_loop(0, EMB_D, dbody, 0)

        fetch_idx(0, idxA, isemA)
        fetch_idx(1, idxB, isemB)

        def body(j, carry):
            p0 = 2 * j
            wait_idx(p0, idxA, isemA)

            @pl.when(p0 >= 2)
            def _():
                wait_put(p0 - 2, rowsA, osemA)

            fill(idxA, rowsA)
            put(p0, rowsA, osemA)

            @pl.when(p0 + 2 < s)
            def _():
                fetch_idx(p0 + 2, idxA, isemA)

            wait_idx(p0 + 1, idxB, isemB)

            @pl.when(p0 >= 2)
            def _():
                wait_put(p0 - 1, rowsB, osemB)

            fill(idxB, rowsB)
            put(p0 + 1, rowsB, osemB)

            @pl.when(p0 + 3 < s)
            def _():
                fetch_idx(p0 + 3, idxB, isemB)

            return carry

        lax.fori_loop(0, s // 2, body, 0)
        wait_put(s - 2, rowsA, osemA)
        wait_put(s - 1, rowsB, osemB)

    return lookup


def kernel(type_indices, embedding_weight):
    b, s = type_indices.shape
    idxt = type_indices.T.astype(jnp.int32)
    wsplat = jnp.broadcast_to(
        embedding_weight[:, :, None], (N_TYPES, EMB_D, LANES))
    out_t = _make_lookup(b, s)(wsplat, idxt)
    return jnp.transpose(out_t, (2, 0, 1))


# speedup vs baseline: 15.2774x; 2.1664x over previous
import functools

import jax
import jax.numpy as jnp
from jax import lax
from jax.experimental import pallas as pl
from jax.experimental.pallas import tpu as pltpu
from jax.experimental.pallas import tpu_sc as plsc

N_TYPES = 3
EMB_D = 64
LANES = 16


@functools.lru_cache(maxsize=None)
def _make_lookup(nb: int, s: int):
    info = plsc.get_sparse_core_info()
    nw = info.num_cores * info.num_subcores
    b_per_w = nb // nw
    ngrp = b_per_w // LANES
    assert nb % (nw * LANES) == 0 and s % 2 == 0

    mesh = plsc.VectorSubcoreMesh(core_axis_name="c", subcore_axis_name="s")

    @functools.partial(
        pl.kernel,
        mesh=mesh,
        compiler_params=pltpu.CompilerParams(use_tc_tiling_on_sc=False),
        out_type=jax.ShapeDtypeStruct(
            (s, EMB_D // 8, nb // 128, 8, 128), jnp.float32),
        scratch_types=[
            pltpu.VMEM((N_TYPES, EMB_D, LANES), jnp.float32),
            pltpu.VMEM((b_per_w,), jnp.int32),
            pltpu.VMEM((b_per_w,), jnp.int32),
            pltpu.VMEM((EMB_D // 8, b_per_w // 128, 8, 128), jnp.float32),
            pltpu.VMEM((EMB_D // 8, b_per_w // 128, 8, 128), jnp.float32),
            pltpu.SemaphoreType.DMA,
            pltpu.SemaphoreType.DMA,
            pltpu.SemaphoreType.DMA,
            pltpu.SemaphoreType.DMA,
        ],
    )
    def lookup(wsplat_hbm, idxt_hbm, out_hbm, wsplat_v, idxA, idxB,
               rowsA, rowsB, isemA, isemB, osemA, osemB):
        wid = lax.axis_index("s") * info.num_cores + lax.axis_index("c")
        bbase = wid * b_per_w
        tcbase = wid * (b_per_w // 128)

        pltpu.sync_copy(wsplat_hbm, wsplat_v)

        def fetch_idx(p, idx_v, isem):
            pltpu.async_copy(idxt_hbm.at[p, pl.ds(bbase, b_per_w)], idx_v, isem)

        def wait_idx(p, idx_v, isem):
            pltpu.make_async_copy(
                idxt_hbm.at[p, pl.ds(bbase, b_per_w)], idx_v, isem).wait()

        def out_slice(p):
            return out_hbm.at[p, pl.ds(0, EMB_D // 8),
                              pl.ds(tcbase, b_per_w // 128)]

        def put(p, rows, osem):
            pltpu.async_copy(rows, out_slice(p), osem)

        def wait_put(p, rows, osem):
            pltpu.make_async_copy(rows, out_slice(p), osem).wait()

        def fill(idx_v, rows):
            for gb in range(ngrp // 4):
                ms = []
                for i in range(4):
                    v = idx_v[pl.ds(LANES * (4 * gb + i), LANES)]
                    ms.append((v == 0, v == 1))

                def dbody(d, carry, gb=gb, ms=ms):
                    w0 = wsplat_v[0, d]
                    w1 = wsplat_v[1, d]
                    w2 = wsplat_v[2, d]
                    tr = lax.shift_right_logical(d, 3)
                    dm = d & 7
                    for i in range(4):
                        g = 4 * gb + i
                        m0, m1 = ms[i]
                        rows[tr, g // 8, dm, pl.ds(LANES * (g % 8), LANES)] = \
                            jnp.where(m0, w0, jnp.where(m1, w1, w2))
                    return carry

                lax.fori_loop(0, EMB_D, dbody, 0)

        fetch_idx(0, idxA, isemA)
        fetch_idx(1, idxB, isemB)

        def body(j, carry):
            p0 = 2 * j
            wait_idx(p0, idxA, isemA)

            @pl.when(p0 >= 2)
            def _():
                wait_put(p0 - 2, rowsA, osemA)

            fill(idxA, rowsA)
            put(p0, rowsA, osemA)

            @pl.when(p0 + 2 < s)
            def _():
                fetch_idx(p0 + 2, idxA, isemA)

            wait_idx(p0 + 1, idxB, isemB)

            @pl.when(p0 >= 2)
            def _():
                wait_put(p0 - 1, rowsB, osemB)

            fill(idxB, rowsB)
            put(p0 + 1, rowsB, osemB)

            @pl.when(p0 + 3 < s)
            def _():
                fetch_idx(p0 + 3, idxB, isemB)

            return carry

        lax.fori_loop(0, s // 2, body, 0)
        wait_put(s - 2, rowsA, osemA)
        wait_put(s - 1, rowsB, osemB)

    return lookup


def kernel(type_indices, embedding_weight):
    b, s = type_indices.shape
    idxt = type_indices.T.astype(jnp.int32)
    wsplat = jnp.broadcast_to(
        embedding_weight[:, :, None], (N_TYPES, EMB_D, LANES))
    out5 = _make_lookup(b, s)(wsplat, idxt)
    out_t = jnp.transpose(out5, (0, 1, 3, 2, 4)).reshape(s, EMB_D, b)
    return jnp.transpose(out_t, (2, 0, 1))


# speedup vs baseline: 22.3141x; 1.4606x over previous
import functools

import jax
import jax.numpy as jnp
from jax import lax
from jax.experimental import pallas as pl
from jax.experimental.pallas import tpu as pltpu
from jax.experimental.pallas import tpu_sc as plsc

N_TYPES = 3
EMB_D = 64
LANES = 16


@functools.lru_cache(maxsize=None)
def _make_lookup(nb: int, s: int):
    info = plsc.get_sparse_core_info()
    nw = info.num_cores * info.num_subcores
    b_per_w = nb // nw
    ngrp = b_per_w // LANES
    assert nb % (nw * LANES) == 0 and s % 2 == 0

    mesh = plsc.VectorSubcoreMesh(core_axis_name="c", subcore_axis_name="s")

    @functools.partial(
        pl.kernel,
        mesh=mesh,
        compiler_params=pltpu.CompilerParams(use_tc_tiling_on_sc=False),
        out_type=jax.ShapeDtypeStruct(
            (s, EMB_D // 8, nb // 128, 8, 128), jnp.float32),
        scratch_types=[
            pltpu.VMEM((N_TYPES, EMB_D, LANES), jnp.float32),
            pltpu.VMEM((b_per_w,), jnp.int32),
            pltpu.VMEM((b_per_w,), jnp.int32),
            pltpu.VMEM((EMB_D // 8, b_per_w // 128, 8, 128), jnp.float32),
            pltpu.VMEM((EMB_D // 8, b_per_w // 128, 8, 128), jnp.float32),
            pltpu.SemaphoreType.DMA,
            pltpu.SemaphoreType.DMA,
            pltpu.SemaphoreType.DMA,
            pltpu.SemaphoreType.DMA,
        ],
    )
    def lookup(wsplat_hbm, idxt_hbm, out_hbm, wsplat_v, idxA, idxB,
               rowsA, rowsB, isemA, isemB, osemA, osemB):
        wid = lax.axis_index("s") * info.num_cores + lax.axis_index("c")
        bbase = wid * b_per_w
        tcbase = wid * (b_per_w // 128)

        pltpu.sync_copy(wsplat_hbm, wsplat_v)

        def fetch_idx(p, idx_v, isem):
            pltpu.async_copy(idxt_hbm.at[p, pl.ds(bbase, b_per_w)], idx_v, isem)

        def wait_idx(p, idx_v, isem):
            pltpu.make_async_copy(
                idxt_hbm.at[p, pl.ds(bbase, b_per_w)], idx_v, isem).wait()

        def out_slice(p):
            return out_hbm.at[p, pl.ds(0, EMB_D // 8),
                              pl.ds(tcbase, b_per_w // 128)]

        def put(p, rows, osem):
            pltpu.async_copy(rows, out_slice(p), osem)

        def wait_put(p, rows, osem):
            pltpu.make_async_copy(rows, out_slice(p), osem).wait()

        def fill(idx_v, rows):
            for gb in range(ngrp // 8):
                ms = []
                for i in range(8):
                    v = idx_v[pl.ds(LANES * (8 * gb + i), LANES)]
                    ms.append((v == 0, v == 1))

                def dbody(d, carry, gb=gb, ms=ms):
                    w0 = wsplat_v[0, d]
                    w1 = wsplat_v[1, d]
                    w2 = wsplat_v[2, d]
                    tr = lax.shift_right_logical(d, 3)
                    dm = d & 7
                    for i in range(8):
                        g = 8 * gb + i
                        m0, m1 = ms[i]
                        rows[tr, g // 8, dm, pl.ds(LANES * (g % 8), LANES)] = \
                            jnp.where(m0, w0, jnp.where(m1, w1, w2))
                    return carry

                lax.fori_loop(0, EMB_D, dbody, 0)

        fetch_idx(0, idxA, isemA)
        fetch_idx(1, idxB, isemB)

        def body(j, carry):
            p0 = 2 * j
            wait_idx(p0, idxA, isemA)

            @pl.when(p0 >= 2)
            def _():
                wait_put(p0 - 2, rowsA, osemA)

            fill(idxA, rowsA)
            put(p0, rowsA, osemA)

            @pl.when(p0 + 2 < s)
            def _():
                fetch_idx(p0 + 2, idxA, isemA)

            wait_idx(p0 + 1, idxB, isemB)

            @pl.when(p0 >= 2)
            def _():
                wait_put(p0 - 1, rowsB, osemB)

            fill(idxB, rowsB)
            put(p0 + 1, rowsB, osemB)

            @pl.when(p0 + 3 < s)
            def _():
                fetch_idx(p0 + 3, idxB, isemB)

            return carry

        lax.fori_loop(0, s // 2, body, 0)
        wait_put(s - 2, rowsA, osemA)
        wait_put(s - 1, rowsB, osemB)

    return lookup


def kernel(type_indices, embedding_weight):
    b, s = type_indices.shape
    idxt = type_indices.T.astype(jnp.int32)
    wsplat = jnp.broadcast_to(
        embedding_weight[:, :, None], (N_TYPES, EMB_D, LANES))
    out5 = _make_lookup(b, s)(wsplat, idxt)
    out_t = jnp.transpose(out5, (0, 1, 3, 2, 4)).reshape(s, EMB_D, b)
    return jnp.transpose(out_t, (2, 0, 1))
